# baseline (speedup 1.0000x reference)
import functools

import jax
import jax.numpy as jnp
from jax import lax
from jax.experimental import pallas as pl
from jax.experimental.pallas import tpu as pltpu
from jax.experimental.pallas import tpu_sc as plsc

H = 128
HW = H // 2
N_GRID = 100000
N_MESH = 10000
N_EDGE = 320000

NC = 2
NS = 16
NW = NC * NS
EPW = N_EDGE // NW
CH = 400
NCH = EPW // CH
CHS = 200
NCHS = EPW // CHS
NMP = 10240
RPS = NMP // NS

_f32 = jnp.float32
_bf16 = jnp.bfloat16


def _silu(x):
    return x * (1.0 / (1.0 + jnp.exp(-x)))


def _ln(h, gam, bet):
    mu = jnp.mean(h, axis=-1, keepdims=True)
    var = jnp.mean((h - mu) ** 2, axis=-1, keepdims=True)
    return (h - mu) * lax.rsqrt(var + 1e-5) * gam + bet


def _dot(a, b):
    return jnp.dot(a, b, preferred_element_type=_f32)


def _w_spec(r, c):
    return pl.BlockSpec((r, c), lambda i: (0, 0))


def _row_spec(blk, c):
    return pl.BlockSpec((blk, c), lambda i: (i, 0))


def _grid_embed_body(x, w1, b1, w2, b2, g1, be1, wp, g_out, gp_out):
    h1 = lax.dot_general(x[...], w1[...], (((0,), (0,)), ((), ())),
                         preferred_element_type=_f32)
    g = _ln(_dot(_silu(h1 + b1[...]), w2[...]) + b2[...],
            g1[...], be1[...])
    g_out[...] = g
    gp_out[...] = _dot(g, wp[...])


def _grid_node_body(g_in, w3, b3, w4, b4, g2, be2, gn_out):
    g = g_in[...]
    gn_out[...] = g + _ln(
        _dot(_silu(_dot(g, w3[...]) + b3[...]), w4[...]) + b4[...],
        g2[...], be2[...])


def _mesh_body(x, w1, b1, w2, b2, g1, be1, wp, m_out, mp_out):
    h1 = lax.dot_general(x[...], w1[...], (((0,), (0,)), ((), ())),
                         preferred_element_type=_f32)
    m = _ln(_dot(_silu(h1 + b1[...]), w2[...]) + b2[...],
            g1[...], be1[...])
    m_out[...] = m
    mp_out[...] = _dot(m, wp[...])


def _edge_fused_body(x, gs, md, w1, b1, w2, b2, g1, be1, wp, bp,
                     w2i, b2i, g2i, be2i, eu_out, en_out):
    e1 = lax.dot_general(x[...], w1[...], (((0,), (0,)), ((), ())),
                         preferred_element_type=_f32)
    e = _ln(_dot(_silu(e1 + b1[...]), w2[...]) + b2[...],
            g1[...], be1[...])
    ep = _dot(e, wp[...]) + bp[...]
    h = _silu(gs[...] + md[...] + ep)
    eu = _ln(_dot(h, w2i[...]) + b2i[...], g2i[...], be2i[...])
    eu_out[...] = eu
    en_out[...] = e + eu


def _edge_fused_body_b(x, gs, md, w1, b1, w2, b2, g1, be1, wp, bp,
                       w2i, b2i, g2i, be2i, en_alias, eu_out, en_out):
    _edge_fused_body(x, gs, md, w1, b1, w2, b2, g1, be1, wp, bp,
                     w2i, b2i, g2i, be2i, eu_out, en_out)


def _mesh_upd_body(m, pa, pb, w1m, w1a, b1, w2, b2, g1, be1, mn_out):
    aggr = (pa[0] + pa[1]) + (pb[0] + pb[1])
    mm = m[...]
    h = _silu(_dot(mm, w1m[...]) + _dot(aggr, w1a[...]) + b1[...])
    mn_out[...] = mm + _ln(_dot(h, w2[...]) + b2[...], g1[...], be1[...])


@functools.lru_cache(maxsize=None)
def _sc_mesh():
    return plsc.VectorSubcoreMesh(core_axis_name="c", subcore_axis_name="s",
                                  num_cores=NC, num_subcores=NS)


@functools.lru_cache(maxsize=None)
def _build_sc_gather(ne, ch):
    epw = ne // NW
    nch = epw // ch
    assert epw % ch == 0 and ch % 8 == 0 and epw % 8 == 0

    @functools.partial(
        pl.kernel, mesh=_sc_mesh(),
        out_type=jax.ShapeDtypeStruct((ne, H), _f32),
        scratch_types=[
            pltpu.VMEM((ch,), jnp.int32),
            pltpu.VMEM((ch, H), _f32),
            pltpu.SemaphoreType.DMA,
        ])
    def _sc_gather_kernel(tab_hbm, idx_hbm, out_hbm, idx_v, buf, sem):
        wid = lax.axis_index("s") * NC + lax.axis_index("c")
        base0 = pl.multiple_of(wid * epw, 8)

        def body(i, carry):
            base = pl.multiple_of(base0 + i * ch, 8)
            pltpu.sync_copy(idx_hbm.at[pl.ds(base, ch)], idx_v)
            pltpu.async_copy(tab_hbm.at[idx_v], buf, sem).wait()
            pltpu.sync_copy(buf, out_hbm.at[pl.ds(base, ch)])
            return carry

        lax.fori_loop(0, nch, body, 0)

    return _sc_gather_kernel


@functools.lru_cache(maxsize=None)
def _build_sc_scatter(ne, chs):
    epw = ne // NW
    nchs = epw // chs
    assert epw % chs == 0 and chs % 8 == 0 and epw % 8 == 0

    @functools.partial(
        pl.kernel, mesh=_sc_mesh(),
        out_type=jax.ShapeDtypeStruct((NC * NMP, H), _f32),
        scratch_types=[
            pltpu.VMEM((chs,), jnp.int32),
            pltpu.VMEM((chs, H), _f32),
            pltpu.VMEM_SHARED((NMP, H), _f32),
        ])
    def _sc_scatter_kernel(eupd_hbm, dst_hbm, zero_hbm, out_hbm,
                           idx_d, buf, accum):
        cid = lax.axis_index("c")
        sid = lax.axis_index("s")
        wid = sid * NC + cid
        base0 = pl.multiple_of(wid * epw, 8)

        pltpu.sync_copy(zero_hbm, accum.at[pl.ds(sid * RPS, RPS)])
        plsc.subcore_barrier()

        def body(i, carry):
            base = pl.multiple_of(base0 + i * chs, 8)
            pltpu.sync_copy(dst_hbm.at[pl.ds(base, chs)], idx_d)
            pltpu.sync_copy(eupd_hbm.at[pl.ds(base, chs)], buf)
            pltpu.sync_copy(buf, accum.at[idx_d], add=True)
            return carry

        lax.fori_loop(0, nchs, body, 0)
        plsc.subcore_barrier()

        pltpu.sync_copy(accum.at[pl.ds(sid * RPS, RPS)],
                        out_hbm.at[pl.ds(cid * NMP + sid * RPS, RPS)])

    return _sc_scatter_kernel


def _sc_gather1(tab, idx, ch=200):
    return _build_sc_gather(idx.shape[0], ch)(tab, idx)


def _sc_scatter(e_upd, dst, chs=200):
    zero = jnp.zeros((RPS, H), _f32)
    partials = _build_sc_scatter(e_upd.shape[0], chs)(e_upd, dst, zero)
    return partials.reshape(NC, NMP, H)


def _bf16_to_f32view(x):
    return lax.bitcast_convert_type(x.reshape(x.shape[0], HW, 2), _f32)


def _f32view_to_bf16(x):
    return lax.bitcast_convert_type(x, _bf16).reshape(x.shape[0], H)


def _mlp_w(params, name):
    p = params
    return (p[name + '_W1'], p[name + '_b1'].reshape(1, H),
            p[name + '_W2'], p[name + '_b2'].reshape(1, H),
            p[name + '_g'].reshape(1, H), p[name + '_beta'].reshape(1, H))


def kernel(grid_nfeat, mesh_nfeat, edge_index, grid2mesh_efeat, params):
    src = edge_index[0].astype(jnp.int32)
    dst = edge_index[1].astype(jnp.int32)

    w1e = params['ie_edge_W1']
    w1e_g, w1e_m, w1e_e = w1e[:H], w1e[H:2 * H], w1e[2 * H:]
    b1e = params['ie_edge_b1'].reshape(1, H)
    w1n = params['ie_node_W1']
    w1n_m, w1n_a = w1n[:H], w1n[H:]

    GB = 2048
    GN = (N_GRID + GB - 1) // GB
    gw = _mlp_w(params, 'emb_grid')
    gnw = _mlp_w(params, 'grid_node')
    g, g1p = pl.pallas_call(
        _grid_embed_body,
        grid=(GN,),
        in_specs=[pl.BlockSpec((3, GB), lambda i: (0, i)),
                  _w_spec(3, H), _w_spec(1, H), _w_spec(H, H), _w_spec(1, H),
                  _w_spec(1, H), _w_spec(1, H), _w_spec(H, H)],
        out_specs=[_row_spec(GB, H)] * 2,
        out_shape=[jax.ShapeDtypeStruct((N_GRID, H), _f32)] * 2,
    )(grid_nfeat.T, gw[0], gw[1], gw[2], gw[3], gw[4], gw[5], w1e_g)
    g_new = pl.pallas_call(
        _grid_node_body,
        grid=(GN,),
        in_specs=[_row_spec(GB, H),
                  _w_spec(H, H), _w_spec(1, H), _w_spec(H, H), _w_spec(1, H),
                  _w_spec(1, H), _w_spec(1, H)],
        out_specs=_row_spec(GB, H),
        out_shape=jax.ShapeDtypeStruct((N_GRID, H), _f32),
    )(g, gnw[0], gnw[1], gnw[2], gnw[3], gnw[4], gnw[5])

    MB = 2048
    MN = (N_MESH + MB - 1) // MB
    mw = _mlp_w(params, 'emb_mesh')
    m, m1p = pl.pallas_call(
        _mesh_body,
        grid=(MN,),
        in_specs=[pl.BlockSpec((3, MB), lambda i: (0, i)),
                  _w_spec(3, H), _w_spec(1, H), _w_spec(H, H), _w_spec(1, H),
                  _w_spec(1, H), _w_spec(1, H), _w_spec(H, H)],
        out_specs=[_row_spec(MB, H)] * 2,
        out_shape=[jax.ShapeDtypeStruct((N_MESH, H), _f32),
                   jax.ShapeDtypeStruct((N_MESH, H), _f32)],
    )(mesh_nfeat.T, mw[0], mw[1], mw[2], mw[3], mw[4], mw[5], w1e_m)

    E2 = N_EDGE // 2
    EB = 3200
    EN2 = E2 // EB
    eft = grid2mesh_efeat.T
    ew = _mlp_w(params, 'emb_edge')
    iew = (params['ie_edge_W2'], params['ie_edge_b2'].reshape(1, H),
           params['ie_edge_g'].reshape(1, H),
           params['ie_edge_beta'].reshape(1, H))

    srcA, srcB = src[:E2], src[E2:]
    dstA, dstB = dst[:E2], dst[E2:]
    mdA = _sc_gather1(m1p, dstA)
    gsA = _sc_gather1(g1p, srcA)
    mdB = _sc_gather1(m1p, dstB)
    gsB = _sc_gather1(g1p, srcB)

    edge_w = (ew[0], ew[1], ew[2], ew[3], ew[4], ew[5], w1e_e, b1e,
              iew[0], iew[1], iew[2], iew[3])
    edge_w_specs = [_w_spec(4, H), _w_spec(1, H), _w_spec(H, H),
                    _w_spec(1, H), _w_spec(1, H), _w_spec(1, H),
                    _w_spec(H, H), _w_spec(1, H),
                    _w_spec(H, H), _w_spec(1, H), _w_spec(1, H),
                    _w_spec(1, H)]

    e_updA, e_newA = pl.pallas_call(
        _edge_fused_body,
        grid=(EN2,),
        in_specs=[pl.BlockSpec((4, EB), lambda i: (0, i)),
                  _row_spec(EB, H), _row_spec(EB, H)]
                 + edge_w_specs,
        out_specs=[_row_spec(EB, H),
                   pl.BlockSpec((EB, H), lambda i: (i, 0))],
        out_shape=[jax.ShapeDtypeStruct((E2, H), _f32),
                   jax.ShapeDtypeStruct((N_EDGE, H), _f32)],
    )(eft, gsA, mdA, *edge_w)

    partialsA = _sc_scatter(e_updA, dstA)

    e_updB, e_new = pl.pallas_call(
        _edge_fused_body_b,
        grid=(EN2,),
        in_specs=[pl.BlockSpec((4, EB), lambda i: (0, i + EN2)),
                  _row_spec(EB, H), _row_spec(EB, H)]
                 + edge_w_specs
                 + [pl.BlockSpec(memory_space=pl.ANY)],
        out_specs=[_row_spec(EB, H),
                   pl.BlockSpec((EB, H), lambda i: (i + EN2, 0))],
        out_shape=[jax.ShapeDtypeStruct((E2, H), _f32),
                   jax.ShapeDtypeStruct((N_EDGE, H), _f32)],
        input_output_aliases={15: 1},
    )(eft, gsB, mdB, *edge_w, e_newA)

    partialsB = _sc_scatter(e_updB, dstB)

    MU, MUN = 2000, N_MESH // 2000
    nw = _mlp_w(params, 'ie_node')
    m_new = pl.pallas_call(
        _mesh_upd_body,
        grid=(MUN,),
        in_specs=[_row_spec(MU, H),
                  pl.BlockSpec((NC, MU, H), lambda i: (0, i, 0)),
                  pl.BlockSpec((NC, MU, H), lambda i: (0, i, 0)),
                  _w_spec(H, H), _w_spec(H, H), _w_spec(1, H),
                  _w_spec(H, H), _w_spec(1, H), _w_spec(1, H), _w_spec(1, H)],
        out_specs=_row_spec(MU, H),
        out_shape=jax.ShapeDtypeStruct((N_MESH, H), _f32),
    )(m, partialsA, partialsB, w1n_m, w1n_a, nw[1], nw[2], nw[3], nw[4],
      nw[5])

    return (g_new, m_new, e_new)


# speedup vs baseline: 1.0536x; 1.0536x over previous
import functools

import jax
import jax.numpy as jnp
from jax import lax
from jax.experimental import pallas as pl
from jax.experimental.pallas import tpu as pltpu
from jax.experimental.pallas import tpu_sc as plsc

H = 128
HW = H // 2
N_GRID = 100000
N_MESH = 10000
N_EDGE = 320000

NC = 2
NS = 16
NW = NC * NS
EPW = N_EDGE // NW
CH = 400
NCH = EPW // CH
CHS = 200
NCHS = EPW // CHS
NMP = 10240
RPS = NMP // NS

_f32 = jnp.float32
_bf16 = jnp.bfloat16


def _silu(x):
    return x * (1.0 / (1.0 + jnp.exp(-x)))


def _ln(h, gam, bet):
    mu = jnp.mean(h, axis=-1, keepdims=True)
    var = jnp.mean((h - mu) ** 2, axis=-1, keepdims=True)
    return (h - mu) * lax.rsqrt(var + 1e-5) * gam + bet


def _dot(a, b):
    return jnp.dot(a, b, preferred_element_type=_f32)


def _w_spec(r, c):
    return pl.BlockSpec((r, c), lambda i: (0, 0))


def _row_spec(blk, c):
    return pl.BlockSpec((blk, c), lambda i: (i, 0))


def _grid_embed_body(x, w1, b1, w2, b2, g1, be1, wp, g_out, gp_out):
    h1 = lax.dot_general(x[...], w1[...], (((0,), (0,)), ((), ())),
                         preferred_element_type=_f32)
    g = _ln(_dot(_silu(h1 + b1[...]), w2[...]) + b2[...],
            g1[...], be1[...])
    g_out[...] = g
    gp_out[...] = _dot(g, wp[...])


def _grid_node_body(g_in, w3, b3, w4, b4, g2, be2, gn_out):
    g = g_in[...]
    gn_out[...] = g + _ln(
        _dot(_silu(_dot(g, w3[...]) + b3[...]), w4[...]) + b4[...],
        g2[...], be2[...])


def _mesh_body(x, w1, b1, w2, b2, g1, be1, wp, m_out, mp_out):
    h1 = lax.dot_general(x[...], w1[...], (((0,), (0,)), ((), ())),
                         preferred_element_type=_f32)
    m = _ln(_dot(_silu(h1 + b1[...]), w2[...]) + b2[...],
            g1[...], be1[...])
    m_out[...] = m
    mp_out[...] = _dot(m, wp[...])


def _edge_fused_body(x, gs, md, w1, b1, w2, b2, g1, be1, wp, bp,
                     w2i, b2i, g2i, be2i, eu_out, en_out):
    e1 = lax.dot_general(x[...], w1[...], (((0,), (0,)), ((), ())),
                         preferred_element_type=_f32)
    e = _ln(_dot(_silu(e1 + b1[...]), w2[...]) + b2[...],
            g1[...], be1[...])
    ep = _dot(e, wp[...]) + bp[...]
    h = _silu(gs[...] + md[...] + ep)
    eu = _ln(_dot(h, w2i[...]) + b2i[...], g2i[...], be2i[...])
    eu_out[...] = eu
    en_out[...] = e + eu


def _edge_fused_body_b(x, gs, md, w1, b1, w2, b2, g1, be1, wp, bp,
                       w2i, b2i, g2i, be2i, en_alias, eu_out, en_out):
    _edge_fused_body(x, gs, md, w1, b1, w2, b2, g1, be1, wp, bp,
                     w2i, b2i, g2i, be2i, eu_out, en_out)


def _mesh_upd_body(m, pa, pb, w1m, w1a, b1, w2, b2, g1, be1, mn_out):
    aggr = (pa[0] + pa[1]) + (pb[0] + pb[1])
    mm = m[...]
    h = _silu(_dot(mm, w1m[...]) + _dot(aggr, w1a[...]) + b1[...])
    mn_out[...] = mm + _ln(_dot(h, w2[...]) + b2[...], g1[...], be1[...])


@functools.lru_cache(maxsize=None)
def _sc_mesh():
    return plsc.VectorSubcoreMesh(core_axis_name="c", subcore_axis_name="s",
                                  num_cores=NC, num_subcores=NS)


@functools.lru_cache(maxsize=None)
def _build_sc_gather2(ne, ch):
    epw = ne // NW
    nch = epw // ch
    assert epw % ch == 0 and ch % 8 == 0 and epw % 8 == 0

    @functools.partial(
        pl.kernel, mesh=_sc_mesh(),
        out_type=(jax.ShapeDtypeStruct((ne, H), _f32),
                  jax.ShapeDtypeStruct((ne, H), _f32)),
        scratch_types=[
            pltpu.VMEM((ch,), jnp.int32), pltpu.VMEM((ch,), jnp.int32),
            pltpu.VMEM((ch, H), _f32), pltpu.VMEM((ch, H), _f32),
            pltpu.SemaphoreType.DMA, pltpu.SemaphoreType.DMA,
        ])
    def _sc_gather_kernel(g1_hbm, m1_hbm, src_hbm, dst_hbm, gs_hbm, md_hbm,
                          idx_s, idx_d, buf_s, buf_d, sem_s, sem_d):
        wid = lax.axis_index("s") * NC + lax.axis_index("c")
        base0 = pl.multiple_of(wid * epw, 8)

        def body(i, carry):
            base = pl.multiple_of(base0 + i * ch, 8)
            pltpu.sync_copy(src_hbm.at[pl.ds(base, ch)], idx_s)
            pltpu.sync_copy(dst_hbm.at[pl.ds(base, ch)], idx_d)
            cs = pltpu.async_copy(g1_hbm.at[idx_s], buf_s, sem_s)
            cd = pltpu.async_copy(m1_hbm.at[idx_d], buf_d, sem_d)
            cs.wait()
            cd.wait()
            pltpu.sync_copy(buf_s, gs_hbm.at[pl.ds(base, ch)])
            pltpu.sync_copy(buf_d, md_hbm.at[pl.ds(base, ch)])
            return carry

        lax.fori_loop(0, nch, body, 0)

    return _sc_gather_kernel


@functools.lru_cache(maxsize=None)
def _build_sc_scatter(ne, chs):
    epw = ne // NW
    nchs = epw // chs
    assert epw % chs == 0 and chs % 8 == 0 and epw % 8 == 0

    @functools.partial(
        pl.kernel, mesh=_sc_mesh(),
        out_type=jax.ShapeDtypeStruct((NC * NMP, H), _f32),
        scratch_types=[
            pltpu.VMEM((chs,), jnp.int32),
            pltpu.VMEM((chs, H), _f32),
            pltpu.VMEM_SHARED((NMP, H), _f32),
        ])
    def _sc_scatter_kernel(eupd_hbm, dst_hbm, zero_hbm, out_hbm,
                           idx_d, buf, accum):
        cid = lax.axis_index("c")
        sid = lax.axis_index("s")
        wid = sid * NC + cid
        base0 = pl.multiple_of(wid * epw, 8)

        pltpu.sync_copy(zero_hbm, accum.at[pl.ds(sid * RPS, RPS)])
        plsc.subcore_barrier()

        def body(i, carry):
            base = pl.multiple_of(base0 + i * chs, 8)
            pltpu.sync_copy(dst_hbm.at[pl.ds(base, chs)], idx_d)
            pltpu.sync_copy(eupd_hbm.at[pl.ds(base, chs)], buf)
            pltpu.sync_copy(buf, accum.at[idx_d], add=True)
            return carry

        lax.fori_loop(0, nchs, body, 0)
        plsc.subcore_barrier()

        pltpu.sync_copy(accum.at[pl.ds(sid * RPS, RPS)],
                        out_hbm.at[pl.ds(cid * NMP + sid * RPS, RPS)])

    return _sc_scatter_kernel


def _sc_gather2(g1p, m1p, src, dst, ch=200):
    return _build_sc_gather2(src.shape[0], ch)(g1p, m1p, src, dst)


def _sc_scatter(e_upd, dst, chs=200):
    zero = jnp.zeros((RPS, H), _f32)
    partials = _build_sc_scatter(e_upd.shape[0], chs)(e_upd, dst, zero)
    return partials.reshape(NC, NMP, H)


def _bf16_to_f32view(x):
    return lax.bitcast_convert_type(x.reshape(x.shape[0], HW, 2), _f32)


def _f32view_to_bf16(x):
    return lax.bitcast_convert_type(x, _bf16).reshape(x.shape[0], H)


def _mlp_w(params, name):
    p = params
    return (p[name + '_W1'], p[name + '_b1'].reshape(1, H),
            p[name + '_W2'], p[name + '_b2'].reshape(1, H),
            p[name + '_g'].reshape(1, H), p[name + '_beta'].reshape(1, H))


def kernel(grid_nfeat, mesh_nfeat, edge_index, grid2mesh_efeat, params):
    src = edge_index[0].astype(jnp.int32)
    dst = edge_index[1].astype(jnp.int32)

    w1e = params['ie_edge_W1']
    w1e_g, w1e_m, w1e_e = w1e[:H], w1e[H:2 * H], w1e[2 * H:]
    b1e = params['ie_edge_b1'].reshape(1, H)
    w1n = params['ie_node_W1']
    w1n_m, w1n_a = w1n[:H], w1n[H:]

    GB = 2048
    GN = (N_GRID + GB - 1) // GB
    gw = _mlp_w(params, 'emb_grid')
    gnw = _mlp_w(params, 'grid_node')
    g, g1p = pl.pallas_call(
        _grid_embed_body,
        grid=(GN,),
        in_specs=[pl.BlockSpec((3, GB), lambda i: (0, i)),
                  _w_spec(3, H), _w_spec(1, H), _w_spec(H, H), _w_spec(1, H),
                  _w_spec(1, H), _w_spec(1, H), _w_spec(H, H)],
        out_specs=[_row_spec(GB, H)] * 2,
        out_shape=[jax.ShapeDtypeStruct((N_GRID, H), _f32)] * 2,
    )(grid_nfeat.T, gw[0], gw[1], gw[2], gw[3], gw[4], gw[5], w1e_g)
    g_new = pl.pallas_call(
        _grid_node_body,
        grid=(GN,),
        in_specs=[_row_spec(GB, H),
                  _w_spec(H, H), _w_spec(1, H), _w_spec(H, H), _w_spec(1, H),
                  _w_spec(1, H), _w_spec(1, H)],
        out_specs=_row_spec(GB, H),
        out_shape=jax.ShapeDtypeStruct((N_GRID, H), _f32),
    )(g, gnw[0], gnw[1], gnw[2], gnw[3], gnw[4], gnw[5])

    MB = 2048
    MN = (N_MESH + MB - 1) // MB
    mw = _mlp_w(params, 'emb_mesh')
    m, m1p = pl.pallas_call(
        _mesh_body,
        grid=(MN,),
        in_specs=[pl.BlockSpec((3, MB), lambda i: (0, i)),
                  _w_spec(3, H), _w_spec(1, H), _w_spec(H, H), _w_spec(1, H),
                  _w_spec(1, H), _w_spec(1, H), _w_spec(H, H)],
        out_specs=[_row_spec(MB, H)] * 2,
        out_shape=[jax.ShapeDtypeStruct((N_MESH, H), _f32),
                   jax.ShapeDtypeStruct((N_MESH, H), _f32)],
    )(mesh_nfeat.T, mw[0], mw[1], mw[2], mw[3], mw[4], mw[5], w1e_m)

    E2 = N_EDGE // 2
    EB = 3200
    EN2 = E2 // EB
    eft = grid2mesh_efeat.T
    ew = _mlp_w(params, 'emb_edge')
    iew = (params['ie_edge_W2'], params['ie_edge_b2'].reshape(1, H),
           params['ie_edge_g'].reshape(1, H),
           params['ie_edge_beta'].reshape(1, H))

    srcA, srcB = src[:E2], src[E2:]
    dstA, dstB = dst[:E2], dst[E2:]
    gsA, mdA = _sc_gather2(g1p, m1p, srcA, dstA)
    gsB, mdB = _sc_gather2(g1p, m1p, srcB, dstB)

    edge_w = (ew[0], ew[1], ew[2], ew[3], ew[4], ew[5], w1e_e, b1e,
              iew[0], iew[1], iew[2], iew[3])
    edge_w_specs = [_w_spec(4, H), _w_spec(1, H), _w_spec(H, H),
                    _w_spec(1, H), _w_spec(1, H), _w_spec(1, H),
                    _w_spec(H, H), _w_spec(1, H),
                    _w_spec(H, H), _w_spec(1, H), _w_spec(1, H),
                    _w_spec(1, H)]

    e_updA, e_newA = pl.pallas_call(
        _edge_fused_body,
        grid=(EN2,),
        in_specs=[pl.BlockSpec((4, EB), lambda i: (0, i)),
                  _row_spec(EB, H), _row_spec(EB, H)]
                 + edge_w_specs,
        out_specs=[_row_spec(EB, H),
                   pl.BlockSpec((EB, H), lambda i: (i, 0))],
        out_shape=[jax.ShapeDtypeStruct((E2, H), _f32),
                   jax.ShapeDtypeStruct((N_EDGE, H), _f32)],
    )(eft, gsA, mdA, *edge_w)

    partialsA = _sc_scatter(e_updA, dstA)

    e_updB, e_new = pl.pallas_call(
        _edge_fused_body_b,
        grid=(EN2,),
        in_specs=[pl.BlockSpec((4, EB), lambda i: (0, i + EN2)),
                  _row_spec(EB, H), _row_spec(EB, H)]
                 + edge_w_specs
                 + [pl.BlockSpec(memory_space=pl.ANY)],
        out_specs=[_row_spec(EB, H),
                   pl.BlockSpec((EB, H), lambda i: (i + EN2, 0))],
        out_shape=[jax.ShapeDtypeStruct((E2, H), _f32),
                   jax.ShapeDtypeStruct((N_EDGE, H), _f32)],
        input_output_aliases={15: 1},
    )(eft, gsB, mdB, *edge_w, e_newA)

    partialsB = _sc_scatter(e_updB, dstB)

    MU, MUN = 2000, N_MESH // 2000
    nw = _mlp_w(params, 'ie_node')
    m_new = pl.pallas_call(
        _mesh_upd_body,
        grid=(MUN,),
        in_specs=[_row_spec(MU, H),
                  pl.BlockSpec((NC, MU, H), lambda i: (0, i, 0)),
                  pl.BlockSpec((NC, MU, H), lambda i: (0, i, 0)),
                  _w_spec(H, H), _w_spec(H, H), _w_spec(1, H),
                  _w_spec(H, H), _w_spec(1, H), _w_spec(1, H), _w_spec(1, H)],
        out_specs=_row_spec(MU, H),
        out_shape=jax.ShapeDtypeStruct((N_MESH, H), _f32),
    )(m, partialsA, partialsB, w1n_m, w1n_a, nw[1], nw[2], nw[3], nw[4],
      nw[5])

    return (g_new, m_new, e_new)
